# Initial kernel scaffold; baseline (speedup 1.0000x reference)
import jax
import jax.numpy as jnp
from jax.experimental import pallas as pl


def kernel(sparse_inputs, dense_inputs, seq0, seq1, hist0, hist1, lin_emb, fm_emb, seq_emb0, seq_emb1, W_ld, b_ld, W1, b1, W2, b2, W3, b3, W4, b4, Wf, bf, Wl, bl):
    raise NotImplementedError("write your pallas kernel here")


# speedup vs baseline: 3.4147x; 3.4147x over previous
import functools

import jax
import jax.numpy as jnp
from jax import lax
from jax.experimental import pallas as pl
from jax.experimental.pallas import tpu as pltpu
from jax.experimental.pallas import tpu_sc as plsc

B = 4096
N_SPARSE = 26
VOCAB = 100000
EMB = 16
N_DENSE = 13
SEQ_LEN = 50

NC = 2
NS = 16
NW = NC * NS
BPW = B // NW
CHUNK = 16
NCHUNK = BPW // CHUNK


def _sc_gather_body(fm_flat, lin_flat, se0, se1,
                    fmidx_h, linridx_h, s0_h, s1_h, h0_h, h1_h,
                    rows_out, pooled_out, lin_out,
                    fmidx_v, linridx_v, sidx_v,
                    rows_v, linscal_v, srows_v, pooled_v, linbuf_v, sem):
    wid = lax.axis_index("s") * NC + lax.axis_index("c")

    def chunk_body(c, carry):
        base = wid * BPW + c * CHUNK

        pltpu.sync_copy(fmidx_h.at[pl.ds(base * N_SPARSE, CHUNK * N_SPARSE)],
                        fmidx_v)
        pltpu.async_copy(fm_flat.at[fmidx_v], rows_v, sem).wait()
        pltpu.sync_copy(rows_v, rows_out.at[pl.ds(base * N_SPARSE,
                                                  CHUNK * N_SPARSE)])

        cbase = (wid * NCHUNK + c) * (CHUNK * N_SPARSE)
        pltpu.sync_copy(linridx_h.at[pl.ds(cbase, CHUNK * N_SPARSE)],
                        linridx_v)
        pltpu.async_copy(lin_flat.at[linridx_v], linscal_v, sem).wait()
        acc = jnp.zeros((16,), jnp.float32)
        for f in range(N_SPARSE):
            acc = acc + linscal_v[pl.ds(f * CHUNK, CHUNK)]
        linbuf_v[pl.ds(c * CHUNK, CHUNK)] = acc

        seq_srcs = ((se0, s0_h), (se1, s1_h), (fm_flat, h0_h), (fm_flat, h1_h))
        for t, (tab, idx_h) in enumerate(seq_srcs):
            pltpu.sync_copy(idx_h.at[pl.ds(base * SEQ_LEN, CHUNK * SEQ_LEN)],
                            sidx_v)
            pltpu.async_copy(tab.at[sidx_v], srows_v, sem).wait()

            def seq_body(tt, accs):
                return tuple(accs[db] + srows_v[db * SEQ_LEN + tt, :]
                             for db in range(CHUNK))

            accs = lax.fori_loop(0, SEQ_LEN, seq_body,
                                 tuple(jnp.zeros((16,), jnp.float32)
                                       for _ in range(CHUNK)))
            for db in range(CHUNK):
                pooled_v[db * 4 + t, :] = accs[db]
        pltpu.sync_copy(pooled_v, pooled_out.at[pl.ds(base * 4, CHUNK * 4)])
        return carry

    lax.fori_loop(0, NCHUNK, chunk_body, 0)
    pltpu.sync_copy(linbuf_v, lin_out.at[pl.ds(wid * BPW, BPW)])


def _sc_gather(fm_flat, lin_flat, se0, se1, fmidx, linridx, s0, s1, h0, h1):
    mesh = plsc.VectorSubcoreMesh(core_axis_name="c", subcore_axis_name="s")
    f = pl.kernel(
        _sc_gather_body,
        out_type=[
            jax.ShapeDtypeStruct((B * N_SPARSE, EMB), jnp.float32),
            jax.ShapeDtypeStruct((B * 4, EMB), jnp.float32),
            jax.ShapeDtypeStruct((B,), jnp.float32),
        ],
        mesh=mesh,
        compiler_params=pltpu.CompilerParams(use_tc_tiling_on_sc=False),
        scratch_types=[
            pltpu.VMEM((CHUNK * N_SPARSE,), jnp.int32),
            pltpu.VMEM((CHUNK * N_SPARSE,), jnp.int32),
            pltpu.VMEM((CHUNK * SEQ_LEN,), jnp.int32),
            pltpu.VMEM((CHUNK * N_SPARSE, EMB), jnp.float32),
            pltpu.VMEM((CHUNK * N_SPARSE,), jnp.float32),
            pltpu.VMEM((CHUNK * SEQ_LEN, EMB), jnp.float32),
            pltpu.VMEM((CHUNK * 4, EMB), jnp.float32),
            pltpu.VMEM((BPW,), jnp.float32),
            pltpu.SemaphoreType.DMA,
        ],
    )
    return f(fm_flat, lin_flat, se0, se1, fmidx, linridx, s0, s1, h0, h1)


def _tc_body(dense_ref, rows_ref, pooled_ref, lin_ref,
             W_ld_ref, W1a_ref, W1b_ref, W1c_ref, b1_ref,
             W2_ref, b2_ref, W3_ref, b3_ref, W4_ref, b4_ref,
             Wf_ref, bf_ref, Wl_ref, bl_ref,
             fin_ref, like_ref):
    dense = dense_ref[...]
    rows = rows_ref[...]
    pooled = pooled_ref[...] * (1.0 / SEQ_LEN)

    dot = functools.partial(jnp.dot, preferred_element_type=jnp.float32)

    def sel(n):
        k = lax.broadcasted_iota(jnp.int32, (n, EMB), 0)
        e = lax.broadcasted_iota(jnp.int32, (n, EMB), 1)
        return (lax.rem(k, EMB) == e).astype(jnp.float32)

    S_rows = sel(N_SPARSE * EMB)
    S_pool = sel(4 * EMB)
    vsum = dot(rows, S_rows) + dot(pooled, S_pool)
    vsq = dot(rows * rows, S_rows) + dot(pooled * pooled, S_pool)
    second = 0.5 * jnp.sum(vsum * vsum - vsq, axis=1, keepdims=True)

    first = dot(dense, W_ld_ref[...]) + lin_ref[...]

    h = dot(dense, W1a_ref[...]) + dot(rows, W1b_ref[...]) \
        + dot(pooled, W1c_ref[...]) + b1_ref[...]
    h = jnp.maximum(h, 0.0)
    h = jnp.maximum(dot(h, W2_ref[...]) + b2_ref[...], 0.0)
    h = jnp.maximum(dot(h, W3_ref[...]) + b3_ref[...], 0.0)
    dnn = dot(h, W4_ref[...]) + b4_ref[...]

    logits = first + second + dnn
    fin_ref[...] = jax.nn.sigmoid(logits * Wf_ref[...] + bf_ref[...])
    like_ref[...] = jax.nn.sigmoid(logits * Wl_ref[...] + bl_ref[...])


def _tc_dense(dense, rows, pooled, lin,
              W_ld, W1a, W1b, W1c, b1, W2, b2, W3, b3, W4, b4, Wf, bf, Wl, bl):
    BS = 512
    grid = (B // BS,)

    def row_spec(cols):
        return pl.BlockSpec((BS, cols), lambda i: (i, 0))

    def full_spec(a):
        return pl.BlockSpec(a.shape, lambda i: (0, 0))

    weights = (W_ld, W1a, W1b, W1c, b1, W2, b2, W3, b3, W4, b4, Wf, bf, Wl, bl)
    return pl.pallas_call(
        _tc_body,
        grid=grid,
        in_specs=[row_spec(N_DENSE), row_spec(N_SPARSE * EMB),
                  row_spec(4 * EMB), row_spec(1)]
                 + [full_spec(w) for w in weights],
        out_specs=[row_spec(1), row_spec(1)],
        out_shape=[jax.ShapeDtypeStruct((B, 1), jnp.float32),
                   jax.ShapeDtypeStruct((B, 1), jnp.float32)],
    )(dense, rows, pooled, lin, *weights)


def kernel(sparse_inputs, dense_inputs, seq0, seq1, hist0, hist1, lin_emb,
           fm_emb, seq_emb0, seq_emb1, W_ld, b_ld, W1, b1, W2, b2, W3, b3,
           W4, b4, Wf, bf, Wl, bl):
    i32 = jnp.int32
    si = sparse_inputs.astype(i32)
    offs = (jnp.arange(N_SPARSE, dtype=i32) * VOCAB)[None, :]
    fm_idx = si + offs

    def perm(a):
        return (a.reshape(NW, NCHUNK, CHUNK, N_SPARSE)
                 .transpose(0, 1, 3, 2).reshape(-1))

    lin_ridx = perm(fm_idx)

    fm_flat = fm_emb.reshape(N_SPARSE * VOCAB, EMB)
    lin_flat = lin_emb.reshape(N_SPARSE * VOCAB)

    s0 = seq0.astype(i32)
    s1 = seq1.astype(i32)
    h0 = hist0.astype(i32)
    h1 = hist1.astype(i32) + VOCAB

    rows_flat, pooled_flat, lin_sum = _sc_gather(
        fm_flat, lin_flat, seq_emb0, seq_emb1, fm_idx.reshape(-1),
        lin_ridx,
        s0.reshape(-1), s1.reshape(-1), h0.reshape(-1), h1.reshape(-1))

    rows = rows_flat.reshape(B, N_SPARSE * EMB)
    pooled = pooled_flat.reshape(B, 4 * EMB)
    lin = lin_sum.reshape(B, 1) + b_ld[0]

    fin, like = _tc_dense(
        dense_inputs, rows, pooled, lin,
        W_ld, W1[:N_DENSE], W1[N_DENSE:N_DENSE + N_SPARSE * EMB],
        W1[N_DENSE + N_SPARSE * EMB:], b1.reshape(1, 200),
        W2, b2.reshape(1, 200), W3, b3.reshape(1, 200), W4, b4.reshape(1, 1),
        Wf, bf.reshape(1, 1), Wl, bl.reshape(1, 1))
    return (fin, like)
